# baseline (speedup 1.0000x reference)
import functools

import jax
import jax.numpy as jnp
from jax import lax
from jax.experimental import pallas as pl
from jax.experimental.pallas import tpu as pltpu
from jax.experimental.pallas import tpu_sc as plsc

B = 64
MAX_NODES = 40
DIM_H = 256
DIM_K = 2
D_EDGE = DIM_H * DIM_K
E_ROWS = B * MAX_NODES * MAX_NODES
N_ROWS = B * MAX_NODES

NUM_CORES = 2
NUM_SUBCORES = 16
NW = NUM_CORES * NUM_SUBCORES
EPW = E_ROWS // NW
NPW = N_ROWS // NW
CHUNK = 64
NCHUNK = EPW // CHUNK
NODE_CHUNK = 16
NODE_NCHUNK = NPW // NODE_CHUNK
LANES = 16

_mesh = plsc.VectorSubcoreMesh(core_axis_name="c", subcore_axis_name="s")


@functools.partial(
    pl.kernel,
    mesh=_mesh,
    out_type=[
        jax.ShapeDtypeStruct((E_ROWS, D_EDGE), jnp.float32),
        jax.ShapeDtypeStruct((N_ROWS, DIM_H), jnp.float32),
    ],
    scratch_types=[
        pltpu.VMEM((EPW,), jnp.int32),
        pltpu.VMEM((CHUNK, D_EDGE), jnp.float32),
        pltpu.VMEM((CHUNK, D_EDGE), jnp.float32),
        pltpu.VMEM((6, N_ROWS), jnp.int32),
        pltpu.VMEM((6, NODE_CHUNK, DIM_H), jnp.float32),
        pltpu.VMEM((NODE_CHUNK, DIM_H), jnp.float32),
        pltpu.SemaphoreType.DMA,
        pltpu.SemaphoreType.DMA,
    ],
)
def _sc_embed(eidx_hbm, nidx_hbm, etab_hbm, ntab_hbm,
              eout_hbm, nout_hbm,
              eidx_v, ebuf0, ebuf1, nidx_v, nbuf_v, nacc_v,
              gsem, ssem):
    wid = lax.axis_index("s") * NUM_CORES + lax.axis_index("c")
    ebase = wid * EPW
    nbase = wid * NPW

    pltpu.sync_copy(eidx_hbm.at[pl.ds(ebase, EPW)], eidx_v)
    pltpu.sync_copy(nidx_hbm, nidx_v)

    def node_round(ch, _):
        off = pl.multiple_of(ch * NODE_CHUNK, NODE_CHUNK)
        cps = [
            pltpu.async_copy(
                ntab_hbm.at[nidx_v.at[f, pl.ds(nbase + off, NODE_CHUNK)]],
                nbuf_v.at[f], gsem)
            for f in range(6)
        ]
        for cp in cps:
            cp.wait()

        def node_row(r, _):
            for cg in range(DIM_H // LANES):
                sl = pl.ds(cg * LANES, LANES)
                acc = nbuf_v[0, r, sl]
                for f in range(1, 6):
                    acc = acc + nbuf_v[f, r, sl]
                nacc_v[r, sl] = acc
            return 0

        lax.fori_loop(0, NODE_CHUNK, node_row, 0)
        pltpu.sync_copy(
            nacc_v, nout_hbm.at[pl.ds(nbase + off, NODE_CHUNK)])
        return 0

    lax.fori_loop(0, NODE_NCHUNK, node_round, 0)

    def e_gather(ci, buf):
        off = pl.multiple_of(ci * CHUNK, CHUNK)
        return pltpu.make_async_copy(
            etab_hbm.at[eidx_v.at[pl.ds(off, CHUNK)]], buf, gsem)

    def e_store(ci, buf):
        off = pl.multiple_of(ebase + ci * CHUNK, CHUNK)
        return pltpu.make_async_copy(buf, eout_hbm.at[pl.ds(off, CHUNK)], ssem)

    e_gather(0, ebuf0).start()

    def pair(p, _):
        c0 = pl.multiple_of(p * 2, 2)
        c1 = c0 + 1
        e_gather(c0, ebuf0).wait()

        @pl.when(p > 0)
        def _():
            e_store(c0 - 1, ebuf1).wait()

        e_gather(c1, ebuf1).start()
        e_store(c0, ebuf0).start()
        e_gather(c1, ebuf1).wait()
        e_store(c0, ebuf0).wait()

        @pl.when(c1 + 1 < NCHUNK)
        def _():
            e_gather(c1 + 1, ebuf0).start()

        e_store(c1, ebuf1).start()
        return 0

    lax.fori_loop(0, NCHUNK // 2, pair, 0)
    e_store(NCHUNK - 1, ebuf1).wait()


def kernel(node_inds, adj_mat_inds, init_hydrogens, init_charge,
           init_is_in_ring, init_is_aromatic, init_chirality,
           n_table, e_table, h_table, charge_table, ring_table,
           arom_table, chir_table):
    eidx = adj_mat_inds.reshape(E_ROWS).astype(jnp.int32)
    nidx = jnp.stack([
        node_inds.reshape(N_ROWS).astype(jnp.int32),
        init_hydrogens.reshape(N_ROWS).astype(jnp.int32) + 22,
        init_charge.reshape(N_ROWS).astype(jnp.int32) + 28,
        init_is_in_ring.reshape(N_ROWS).astype(jnp.int32) + 34,
        init_is_aromatic.reshape(N_ROWS).astype(jnp.int32) + 37,
        init_chirality.reshape(N_ROWS).astype(jnp.int32) + 40,
    ])
    ntab = jnp.concatenate([n_table, h_table, charge_table, ring_table,
                            arom_table, chir_table], axis=0)
    eout, nout = _sc_embed(eidx, nidx, e_table, ntab)
    node_embeddings = nout.reshape(B, MAX_NODES, DIM_H)
    edge_embeddings = eout.reshape(B, MAX_NODES, MAX_NODES, DIM_H, DIM_K)
    return (node_embeddings, edge_embeddings)


# speedup vs baseline: 2.5338x; 2.5338x over previous
import functools

import jax
import jax.numpy as jnp
from jax import lax
from jax.experimental import pallas as pl
from jax.experimental.pallas import tpu as pltpu
from jax.experimental.pallas import tpu_sc as plsc

B = 64
MAX_NODES = 40
DIM_H = 256
DIM_K = 2
D_EDGE = DIM_H * DIM_K
E_ROWS = B * MAX_NODES * MAX_NODES
N_ROWS = B * MAX_NODES

NUM_CORES = 2
NUM_SUBCORES = 16
NW = NUM_CORES * NUM_SUBCORES
EPW = E_ROWS // NW
NPW = N_ROWS // NW
CHUNK = 64
NCHUNK = EPW // CHUNK
NODE_CHUNK = 16
NODE_NCHUNK = NPW // NODE_CHUNK
LANES = 16

_mesh = plsc.VectorSubcoreMesh(core_axis_name="c", subcore_axis_name="s")


@functools.partial(
    pl.kernel,
    mesh=_mesh,
    out_type=[
        jax.ShapeDtypeStruct((E_ROWS, DIM_K, DIM_H), jnp.float32),
        jax.ShapeDtypeStruct((N_ROWS, DIM_H), jnp.float32),
    ],
    scratch_types=[
        pltpu.VMEM((EPW,), jnp.int32),
        pltpu.VMEM((CHUNK, DIM_K, DIM_H), jnp.float32),
        pltpu.VMEM((CHUNK, DIM_K, DIM_H), jnp.float32),
        pltpu.VMEM((6, N_ROWS), jnp.int32),
        pltpu.VMEM((6, NODE_CHUNK, DIM_H), jnp.float32),
        pltpu.VMEM((NODE_CHUNK, DIM_H), jnp.float32),
        pltpu.SemaphoreType.DMA,
        pltpu.SemaphoreType.DMA,
    ],
)
def _sc_embed(eidx_hbm, nidx_hbm, etab_hbm, ntab_hbm,
              eout_hbm, nout_hbm,
              eidx_v, ebuf0, ebuf1, nidx_v, nbuf_v, nacc_v,
              gsem, ssem):
    wid = lax.axis_index("s") * NUM_CORES + lax.axis_index("c")
    ebase = wid * EPW
    nbase = wid * NPW

    pltpu.sync_copy(eidx_hbm.at[pl.ds(ebase, EPW)], eidx_v)
    pltpu.sync_copy(nidx_hbm, nidx_v)

    def node_round(ch, _):
        off = pl.multiple_of(ch * NODE_CHUNK, NODE_CHUNK)
        cps = [
            pltpu.async_copy(
                ntab_hbm.at[nidx_v.at[f, pl.ds(nbase + off, NODE_CHUNK)]],
                nbuf_v.at[f], gsem)
            for f in range(6)
        ]
        for cp in cps:
            cp.wait()

        def node_row(r, _):
            for cg in range(DIM_H // LANES):
                sl = pl.ds(cg * LANES, LANES)
                acc = nbuf_v[0, r, sl]
                for f in range(1, 6):
                    acc = acc + nbuf_v[f, r, sl]
                nacc_v[r, sl] = acc
            return 0

        lax.fori_loop(0, NODE_CHUNK, node_row, 0)
        pltpu.sync_copy(
            nacc_v, nout_hbm.at[pl.ds(nbase + off, NODE_CHUNK)])
        return 0

    lax.fori_loop(0, NODE_NCHUNK, node_round, 0)

    def e_gather(ci, buf):
        off = pl.multiple_of(ci * CHUNK, CHUNK)
        return pltpu.make_async_copy(
            etab_hbm.at[eidx_v.at[pl.ds(off, CHUNK)]], buf, gsem)

    def e_store(ci, buf):
        off = pl.multiple_of(ebase + ci * CHUNK, CHUNK)
        return pltpu.make_async_copy(buf, eout_hbm.at[pl.ds(off, CHUNK)], ssem)

    e_gather(0, ebuf0).start()

    def pair(p, _):
        c0 = pl.multiple_of(p * 2, 2)
        c1 = c0 + 1
        e_gather(c0, ebuf0).wait()

        @pl.when(p > 0)
        def _():
            e_store(c0 - 1, ebuf1).wait()

        e_gather(c1, ebuf1).start()
        e_store(c0, ebuf0).start()
        e_gather(c1, ebuf1).wait()
        e_store(c0, ebuf0).wait()

        @pl.when(c1 + 1 < NCHUNK)
        def _():
            e_gather(c1 + 1, ebuf0).start()

        e_store(c1, ebuf1).start()
        return 0

    lax.fori_loop(0, NCHUNK // 2, pair, 0)
    e_store(NCHUNK - 1, ebuf1).wait()


def kernel(node_inds, adj_mat_inds, init_hydrogens, init_charge,
           init_is_in_ring, init_is_aromatic, init_chirality,
           n_table, e_table, h_table, charge_table, ring_table,
           arom_table, chir_table):
    eidx = adj_mat_inds.reshape(E_ROWS).astype(jnp.int32)
    etab_de = e_table.reshape(7, DIM_H, DIM_K).transpose(0, 2, 1)
    nidx = jnp.stack([
        node_inds.reshape(N_ROWS).astype(jnp.int32),
        init_hydrogens.reshape(N_ROWS).astype(jnp.int32) + 22,
        init_charge.reshape(N_ROWS).astype(jnp.int32) + 28,
        init_is_in_ring.reshape(N_ROWS).astype(jnp.int32) + 34,
        init_is_aromatic.reshape(N_ROWS).astype(jnp.int32) + 37,
        init_chirality.reshape(N_ROWS).astype(jnp.int32) + 40,
    ])
    ntab = jnp.concatenate([n_table, h_table, charge_table, ring_table,
                            arom_table, chir_table], axis=0)
    eout, nout = _sc_embed(eidx, nidx, etab_de, ntab)
    node_embeddings = nout.reshape(B, MAX_NODES, DIM_H)
    edge_embeddings = (
        eout.reshape(B, MAX_NODES, MAX_NODES, DIM_K, DIM_H)
        .transpose(0, 1, 2, 4, 3))
    return (node_embeddings, edge_embeddings)
